# baseline (speedup 1.0000x reference)
import dataclasses
import functools

import jax
import jax.numpy as jnp
from jax import lax
from jax.experimental import pallas as pl
from jax.experimental.pallas import tpu as pltpu
from jax.experimental.pallas import tpu_sc as plsc

_GENE_BLOCK = 200
_WORKERS_PER_CORE = 4
_NUM_WORKERS = 2 * _WORKERS_PER_CORE
_LANES = 16


def _sc_gather(ctab, idx, dim, n_rows):
    n_idx = idx.shape[0]
    chunk = n_idx // _NUM_WORKERS
    mesh = plsc.VectorSubcoreMesh(core_axis_name="c", subcore_axis_name="s")
    cp = pltpu.CompilerParams()
    if "needs_layout_passes" in pltpu.CompilerParams.__dataclass_fields__:
        cp = dataclasses.replace(cp, needs_layout_passes=False)

    ncols = dim + 1

    tab_len = n_rows * ncols

    @pl.kernel(
        compiler_params=cp,
        out_type=jax.ShapeDtypeStruct((n_idx * ncols,), jnp.float32),
        mesh=mesh,
        scratch_types=[
            pltpu.VMEM((tab_len,), jnp.float32),
            pltpu.VMEM((chunk,), jnp.float32),
            pltpu.VMEM((chunk * ncols,), jnp.float32),
        ],
    )
    def gather_kernel(t_hbm, out_hbm, tab, idxb, outb):
        s = lax.axis_index("s")
        wid = lax.axis_index("c") * _WORKERS_PER_CORE + s
        base = wid * chunk

        @pl.when(s < _WORKERS_PER_CORE)
        def _():
            pltpu.sync_copy(t_hbm.at[pl.ds(0, tab_len)], tab)
            pltpu.sync_copy(t_hbm.at[pl.ds(tab_len + base, chunk)], idxb)
            for k in range(chunk // _LANES):
                idxv = plsc.bitcast(idxb[pl.ds(k * _LANES, _LANES)], jnp.int32)
                pos = (lax.iota(jnp.int32, _LANES) + (k * _LANES)) * ncols
                for d in range(ncols):
                    vals = plsc.load_gather(tab, [idxv + (d * n_rows)])
                    plsc.store_scatter(outb, [pos + d], vals)
            pltpu.sync_copy(outb, out_hbm.at[pl.ds(base * ncols, chunk * ncols)])

    return gather_kernel(jnp.concatenate(
        [ctab, jax.lax.bitcast_convert_type(idx, jnp.float32)])).reshape(n_idx, ncols)


def _tc_body(x_ref, w_ref, o_ref, *, dim: int):
    w = w_ref[...]
    acc = w[:, dim:dim + 1] + x_ref[0] * w[:, 0:1]
    for d in range(1, dim):
        acc = acc + x_ref[d] * w[:, d:d + 1]
    o_ref[...] = acc


def _expression_tc(xt, w6):
    dim, genes, cells = xt.shape
    body = functools.partial(_tc_body, dim=dim)
    out_t = pl.pallas_call(
        body,
        grid=(genes // _GENE_BLOCK,),
        in_specs=[
            pl.BlockSpec((dim, _GENE_BLOCK, cells), lambda j: (0, j, 0)),
            pl.BlockSpec((_GENE_BLOCK, dim + 1), lambda j: (j, 0)),
        ],
        out_specs=pl.BlockSpec((_GENE_BLOCK, cells), lambda j: (j, 0)),
        out_shape=jax.ShapeDtypeStruct((genes, cells), jnp.float32),
    )(xt, w6)
    return out_t


def kernel(cell_gene_embedding, gene_ix, weight1, bias1):
    genes = gene_ix.shape[0]
    pad_to = _NUM_WORKERS * _LANES
    padded = ((genes + pad_to - 1) // pad_to) * pad_to
    idx = jnp.pad(gene_ix, (0, padded - genes))
    n_rows, dim = weight1.shape
    ctab = jnp.concatenate([weight1.T.reshape(-1), bias1.reshape(-1)])
    w6 = _sc_gather(ctab, idx, dim, n_rows)
    xt = jnp.transpose(cell_gene_embedding, (2, 1, 0))
    out_t = _expression_tc(xt, w6)
    return out_t.T


# speedup vs baseline: 1.0396x; 1.0396x over previous
import dataclasses
import functools

import jax
import jax.numpy as jnp
from jax import lax
from jax.experimental import pallas as pl
from jax.experimental.pallas import tpu as pltpu
from jax.experimental.pallas import tpu_sc as plsc

_GENE_BLOCK = 200
_WORKERS_PER_CORE = 4
_NUM_WORKERS = 2 * _WORKERS_PER_CORE
_LANES = 16


def _sc_gather(ctab, idx, dim, n_rows):
    n_idx = idx.shape[0]
    chunk = n_idx // _NUM_WORKERS
    mesh = plsc.VectorSubcoreMesh(core_axis_name="c", subcore_axis_name="s")
    cp = pltpu.CompilerParams()
    if "needs_layout_passes" in pltpu.CompilerParams.__dataclass_fields__:
        cp = dataclasses.replace(cp, needs_layout_passes=False)

    ncols = dim + 1

    @pl.kernel(
        compiler_params=cp,
        out_type=jax.ShapeDtypeStruct((n_idx * ncols,), jnp.float32),
        mesh=mesh,
        scratch_types=[
            pltpu.VMEM((n_rows * ncols,), jnp.float32),
            pltpu.VMEM((chunk,), jnp.int32),
            pltpu.VMEM((chunk * ncols,), jnp.float32),
        ],
    )
    def gather_kernel(t_hbm, i_hbm, out_hbm, tab, idxb, outb):
        s = lax.axis_index("s")
        wid = lax.axis_index("c") * _WORKERS_PER_CORE + s
        base = wid * chunk

        @pl.when(s < _WORKERS_PER_CORE)
        def _():
            pltpu.sync_copy(t_hbm, tab)
            pltpu.sync_copy(i_hbm.at[pl.ds(base, chunk)], idxb)
            for k in range(chunk // _LANES):
                idxv = idxb[pl.ds(k * _LANES, _LANES)]
                pos = (lax.iota(jnp.int32, _LANES) + (k * _LANES)) * ncols
                for d in range(ncols):
                    vals = plsc.load_gather(tab, [idxv + (d * n_rows)])
                    plsc.store_scatter(outb, [pos + d], vals)
            pltpu.sync_copy(outb, out_hbm.at[pl.ds(base * ncols, chunk * ncols)])

    return gather_kernel(ctab, idx).reshape(n_idx, ncols)


def _tc_body(x_ref, w_ref, o_ref, *, dim: int):
    w = w_ref[...]
    acc = w[:, dim:dim + 1] + x_ref[0] * w[:, 0:1]
    for d in range(1, dim):
        acc = acc + x_ref[d] * w[:, d:d + 1]
    o_ref[...] = acc


def _expression_tc(xt, w6):
    dim, genes, cells = xt.shape
    body = functools.partial(_tc_body, dim=dim)
    out_t = pl.pallas_call(
        body,
        grid=(genes // _GENE_BLOCK,),
        in_specs=[
            pl.BlockSpec((dim, _GENE_BLOCK, cells), lambda j: (0, j, 0)),
            pl.BlockSpec((_GENE_BLOCK, dim + 1), lambda j: (j, 0)),
        ],
        out_specs=pl.BlockSpec((_GENE_BLOCK, cells), lambda j: (j, 0)),
        out_shape=jax.ShapeDtypeStruct((genes, cells), jnp.float32),
    )(xt, w6)
    return out_t


def kernel(cell_gene_embedding, gene_ix, weight1, bias1):
    genes = gene_ix.shape[0]
    pad_to = _NUM_WORKERS * _LANES
    padded = ((genes + pad_to - 1) // pad_to) * pad_to
    idx = jnp.pad(gene_ix, (0, padded - genes))
    n_rows, dim = weight1.shape
    ctab = jnp.concatenate([weight1.T.reshape(-1), bias1.reshape(-1)])
    w6 = _sc_gather(ctab, idx, dim, n_rows)
    xt = jnp.transpose(cell_gene_embedding, (2, 1, 0))
    out_t = _expression_tc(xt, w6)
    return out_t.T
